# baseline (speedup 1.0000x reference)
import functools
import math

import jax
import jax.numpy as jnp
from jax import lax
from jax.experimental import pallas as pl
from jax.experimental.pallas import tpu as pltpu
from jax.experimental.pallas import tpu_sc as plsc

D_MODEL = 1024
SCALE = math.sqrt(D_MODEL)

NUM_CORES = 2
NUM_SUBCORES = 16
NUM_WORKERS = NUM_CORES * NUM_SUBCORES
LANES = 16


@functools.partial(jax.jit, static_argnames=("b_total",))
def _scaled_embed(x_flat, table, b_total):
    b_per_w = b_total // NUM_WORKERS
    chunk = 32
    n_chunks = b_per_w // chunk
    mesh = plsc.VectorSubcoreMesh(core_axis_name="c", subcore_axis_name="s")

    @functools.partial(
        pl.kernel,
        mesh=mesh,
        out_type=jax.ShapeDtypeStruct((b_total, D_MODEL), jnp.float32),
        scratch_types=[
            pltpu.VMEM((b_per_w,), jnp.int32),
            pltpu.VMEM((chunk, D_MODEL), jnp.float32),
            pltpu.SemaphoreType.DMA,
        ],
    )
    def k(idx_hbm, table_hbm, out_hbm, idx_v, rows_v, sem):
        wid = lax.axis_index("s") * NUM_CORES + lax.axis_index("c")
        base = wid * b_per_w
        pltpu.sync_copy(idx_hbm.at[pl.ds(base, b_per_w)], idx_v)

        def chunk_body(g, carry):
            pltpu.async_copy(
                table_hbm.at[idx_v.at[pl.ds(g * chunk, chunk)]], rows_v, sem
            ).wait()

            def row_body(r, carry2):
                for j in range(D_MODEL // LANES):
                    sl = pl.ds(j * LANES, LANES)
                    rows_v[r, sl] = rows_v[r, sl] * SCALE
                return carry2

            lax.fori_loop(0, chunk, row_body, 0, unroll=False)
            pltpu.sync_copy(rows_v, out_hbm.at[pl.ds(base + g * chunk, chunk)])
            return carry

        lax.fori_loop(0, n_chunks, chunk_body, 0, unroll=False)

    return k(x_flat, table)


def kernel(x, table):
    b_total = x.shape[0] * x.shape[1]
    x_flat = x.reshape(b_total).astype(jnp.int32)
    out = _scaled_embed(x_flat, table, b_total)
    return out.reshape(x.shape[0], x.shape[1], D_MODEL)


# speedup vs baseline: 1.3381x; 1.3381x over previous
import functools
import math

import jax
import jax.numpy as jnp
from jax import lax
from jax.experimental import pallas as pl
from jax.experimental.pallas import tpu as pltpu
from jax.experimental.pallas import tpu_sc as plsc

D_MODEL = 1024
SCALE = math.sqrt(D_MODEL)

NUM_CORES = 2
NUM_SUBCORES = 16
NUM_WORKERS = NUM_CORES * NUM_SUBCORES
LANES = 16

CHUNK = 32
NBUF = 3
LOOKAHEAD = 2


@functools.partial(jax.jit, static_argnames=("b_total",))
def _scaled_embed(x_flat, table, b_total):
    b_per_w = b_total // NUM_WORKERS
    n_chunks = b_per_w // CHUNK
    mesh = plsc.VectorSubcoreMesh(core_axis_name="c", subcore_axis_name="s")

    @functools.partial(
        pl.kernel,
        mesh=mesh,
        out_type=jax.ShapeDtypeStruct((b_total, D_MODEL), jnp.float32),
        scratch_types=[
            pltpu.VMEM((b_per_w,), jnp.int32),
        ]
        + [pltpu.VMEM((CHUNK, D_MODEL), jnp.float32) for _ in range(NBUF)]
        + [pltpu.SemaphoreType.DMA for _ in range(2 * NBUF)],
    )
    def k(idx_hbm, table_hbm, out_hbm, idx_v, *bufs_and_sems):
        bufs = bufs_and_sems[:NBUF]
        sem_g = bufs_and_sems[NBUF : 2 * NBUF]
        sem_w = bufs_and_sems[2 * NBUF :]

        wid = lax.axis_index("s") * NUM_CORES + lax.axis_index("c")
        base = wid * b_per_w
        pltpu.sync_copy(idx_hbm.at[pl.ds(base, b_per_w)], idx_v)

        gather_h = [None] * n_chunks
        write_h = [None] * n_chunks

        def scale_buf(buf):
            def row_body(r, carry):
                for j in range(D_MODEL // LANES):
                    sl = pl.ds(j * LANES, LANES)
                    buf[r, sl] = buf[r, sl] * SCALE
                return carry

            lax.fori_loop(0, CHUNK, row_body, 0, unroll=False)

        for g in range(n_chunks + LOOKAHEAD):
            if g < n_chunks:
                s = g % NBUF
                if g >= NBUF:
                    write_h[g - NBUF].wait()
                gather_h[g] = pltpu.async_copy(
                    table_hbm.at[idx_v.at[pl.ds(g * CHUNK, CHUNK)]],
                    bufs[s],
                    sem_g[s],
                )
            p = g - LOOKAHEAD
            if p >= 0:
                s = p % NBUF
                gather_h[p].wait()
                scale_buf(bufs[s])
                write_h[p] = pltpu.async_copy(
                    bufs[s],
                    out_hbm.at[pl.ds(base + p * CHUNK, CHUNK)],
                    sem_w[s],
                )
        for p in range(n_chunks - NBUF, n_chunks):
            write_h[p].wait()

    return k(x_flat, table)


def kernel(x, table):
    b_total = x.shape[0] * x.shape[1]
    x_flat = x.reshape(b_total).astype(jnp.int32)
    out = _scaled_embed(x_flat, table, b_total)
    return out.reshape(x.shape[0], x.shape[1], D_MODEL)


# speedup vs baseline: 1.4776x; 1.1042x over previous
import functools
import math

import jax
import jax.numpy as jnp
from jax import lax
from jax.experimental import pallas as pl
from jax.experimental.pallas import tpu as pltpu
from jax.experimental.pallas import tpu_sc as plsc

D_MODEL = 1024
SCALE = math.sqrt(D_MODEL)

NUM_CORES = 2
NUM_SUBCORES = 16
NUM_WORKERS = NUM_CORES * NUM_SUBCORES
LANES = 16

CHUNK = 16
NBUF = 6
LOOKAHEAD = 3


@functools.partial(jax.jit, static_argnames=("b_total",))
def _scaled_embed(x_flat, table, b_total):
    b_per_w = b_total // NUM_WORKERS
    n_chunks = b_per_w // CHUNK
    mesh = plsc.VectorSubcoreMesh(core_axis_name="c", subcore_axis_name="s")

    @functools.partial(
        pl.kernel,
        mesh=mesh,
        out_type=jax.ShapeDtypeStruct((b_total, D_MODEL), jnp.float32),
        scratch_types=[
            pltpu.VMEM((b_per_w,), jnp.int32),
        ]
        + [pltpu.VMEM((CHUNK, D_MODEL), jnp.float32) for _ in range(NBUF)]
        + [pltpu.SemaphoreType.DMA for _ in range(2 * NBUF)],
    )
    def k(idx_hbm, table_hbm, out_hbm, idx_v, *bufs_and_sems):
        bufs = bufs_and_sems[:NBUF]
        sem_g = bufs_and_sems[NBUF : 2 * NBUF]
        sem_w = bufs_and_sems[2 * NBUF :]

        wid = lax.axis_index("s") * NUM_CORES + lax.axis_index("c")
        base = wid * b_per_w
        pltpu.sync_copy(idx_hbm.at[pl.ds(base, b_per_w)], idx_v)

        gather_h = [None] * n_chunks
        write_h = [None] * n_chunks

        def scale_buf(buf):
            def row_body(r, carry):
                for j in range(D_MODEL // LANES):
                    sl = pl.ds(j * LANES, LANES)
                    buf[r, sl] = buf[r, sl] * SCALE
                return carry

            lax.fori_loop(0, CHUNK, row_body, 0, unroll=False)

        for g in range(n_chunks + LOOKAHEAD):
            if g < n_chunks:
                s = g % NBUF
                if g >= NBUF:
                    write_h[g - NBUF].wait()
                gather_h[g] = pltpu.async_copy(
                    table_hbm.at[idx_v.at[pl.ds(g * CHUNK, CHUNK)]],
                    bufs[s],
                    sem_g[s],
                )
            p = g - LOOKAHEAD
            if p >= 0:
                s = p % NBUF
                gather_h[p].wait()
                scale_buf(bufs[s])
                write_h[p] = pltpu.async_copy(
                    bufs[s],
                    out_hbm.at[pl.ds(base + p * CHUNK, CHUNK)],
                    sem_w[s],
                )
        for p in range(n_chunks - NBUF, n_chunks):
            write_h[p].wait()

    return k(x_flat, table)


def kernel(x, table):
    b_total = x.shape[0] * x.shape[1]
    x_flat = x.reshape(b_total).astype(jnp.int32)
    out = _scaled_embed(x_flat, table, b_total)
    return out.reshape(x.shape[0], x.shape[1], D_MODEL)


# speedup vs baseline: 1.4857x; 1.0055x over previous
import functools
import math

import jax
import jax.numpy as jnp
from jax import lax
from jax.experimental import pallas as pl
from jax.experimental.pallas import tpu as pltpu
from jax.experimental.pallas import tpu_sc as plsc

D_MODEL = 1024
SCALE = math.sqrt(D_MODEL)

NUM_CORES = 2
NUM_SUBCORES = 16
NUM_WORKERS = NUM_CORES * NUM_SUBCORES
LANES = 16

CHUNK = 16
NBUF = 7
LOOKAHEAD = 4


@functools.partial(jax.jit, static_argnames=("b_total",))
def _scaled_embed(x_flat, table, b_total):
    b_per_w = b_total // NUM_WORKERS
    n_chunks = b_per_w // CHUNK
    mesh = plsc.VectorSubcoreMesh(core_axis_name="c", subcore_axis_name="s")

    @functools.partial(
        pl.kernel,
        mesh=mesh,
        out_type=jax.ShapeDtypeStruct((b_total, D_MODEL), jnp.float32),
        scratch_types=[
            pltpu.VMEM((b_per_w,), jnp.int32),
        ]
        + [pltpu.VMEM((CHUNK, D_MODEL), jnp.float32) for _ in range(NBUF)]
        + [pltpu.SemaphoreType.DMA for _ in range(2 * NBUF)],
    )
    def k(idx_hbm, table_hbm, out_hbm, idx_v, *bufs_and_sems):
        bufs = bufs_and_sems[:NBUF]
        sem_g = bufs_and_sems[NBUF : 2 * NBUF]
        sem_w = bufs_and_sems[2 * NBUF :]

        wid = lax.axis_index("s") * NUM_CORES + lax.axis_index("c")
        base = wid * b_per_w
        pltpu.sync_copy(idx_hbm.at[pl.ds(base, b_per_w)], idx_v)

        gather_h = [None] * n_chunks
        write_h = [None] * n_chunks

        def scale_buf(buf):
            def row_body(r, carry):
                for j in range(D_MODEL // LANES):
                    sl = pl.ds(j * LANES, LANES)
                    buf[r, sl] = buf[r, sl] * SCALE
                return carry

            lax.fori_loop(0, CHUNK, row_body, 0, unroll=False)

        for g in range(n_chunks + LOOKAHEAD):
            if g < n_chunks:
                s = g % NBUF
                if g >= NBUF:
                    write_h[g - NBUF].wait()
                gather_h[g] = pltpu.async_copy(
                    table_hbm.at[idx_v.at[pl.ds(g * CHUNK, CHUNK)]],
                    bufs[s],
                    sem_g[s],
                )
            p = g - LOOKAHEAD
            if p >= 0:
                s = p % NBUF
                gather_h[p].wait()
                scale_buf(bufs[s])
                write_h[p] = pltpu.async_copy(
                    bufs[s],
                    out_hbm.at[pl.ds(base + p * CHUNK, CHUNK)],
                    sem_w[s],
                )
        for p in range(n_chunks - NBUF, n_chunks):
            write_h[p].wait()

    return k(x_flat, table)


def kernel(x, table):
    b_total = x.shape[0] * x.shape[1]
    x_flat = x.reshape(b_total).astype(jnp.int32)
    out = _scaled_embed(x_flat, table, b_total)
    return out.reshape(x.shape[0], x.shape[1], D_MODEL)
